# Initial kernel scaffold; baseline (speedup 1.0000x reference)
import jax
import jax.numpy as jnp
from jax.experimental import pallas as pl


def kernel(rel_pair_idxs, boxes, rel_labels, pred_rel_logits, pred_threshold):
    raise NotImplementedError("write your pallas kernel here")


# speedup vs baseline: 344.8158x; 344.8158x over previous
import functools

import jax
import jax.numpy as jnp
from jax import lax
from jax.experimental import pallas as pl
from jax.experimental.pallas import tpu as pltpu
from jax.experimental.pallas import tpu_sc as plsc

NUM_CLS = 51
N_BOXES = 128
N_PAIRS = 1024
NC, NS, L = 2, 16, 16
PAIRS_PER_SUB = N_PAIRS // NS
OUT_PER_SUB = N_PAIRS // (NC * NS)
CPAD = 64
BIG_I = 1 << 30


def _body(lg_hbm, a_hbm, b_hbm, x1_hbm, y1_hbm, x2_hbm, y2_hbm, thr_hbm,
          mconf_hbm, final_hbm, clso_hbm,
          lg_v, a_v, b_v, x1_v, y1_v, x2_v, y2_v, thr_v,
          anyrow_v, anyov_v, conf_b, cls_b, valid_b,
          conf_all, cls_all, valid_all,
          t3c_b, t3i_b, t3c_v, t3i_v, mc_b, fin_b, cls_ob,
          anyov_sh, conf_sh, cls_sh, valid_sh, t3c_sh, t3i_sh):
  core = lax.axis_index("c")
  sid = lax.axis_index("s")
  iota = lax.iota(jnp.int32, L)

  pltpu.sync_copy(lg_hbm.at[pl.ds(sid * (PAIRS_PER_SUB * CPAD), PAIRS_PER_SUB * CPAD)], lg_v)
  pltpu.sync_copy(a_hbm.at[pl.ds(sid * PAIRS_PER_SUB, PAIRS_PER_SUB)], a_v)
  pltpu.sync_copy(b_hbm.at[pl.ds(sid * PAIRS_PER_SUB, PAIRS_PER_SUB)], b_v)
  pltpu.sync_copy(x1_hbm, x1_v)
  pltpu.sync_copy(y1_hbm, y1_v)
  pltpu.sync_copy(x2_hbm, x2_v)
  pltpu.sync_copy(y2_hbm, y2_v)
  pltpu.sync_copy(thr_hbm, thr_v)

  @pl.when(sid < (N_BOXES // L))
  def _stage0():
    x1i = x1_v[pl.ds(sid * L, L)]
    y1i = y1_v[pl.ds(sid * L, L)]
    x2i = x2_v[pl.ds(sid * L, L)]
    y2i = y2_v[pl.ds(sid * L, L)]
    i_vec = sid * L + iota

    def jbody(j, acc):
      js = jnp.full((L,), 0, jnp.int32) + j
      x1j = plsc.load_gather(x1_v, [js])
      y1j = plsc.load_gather(y1_v, [js])
      x2j = plsc.load_gather(x2_v, [js])
      y2j = plsc.load_gather(y2_v, [js])
      w = jnp.maximum(jnp.minimum(x2i, x2j) - jnp.maximum(x1i, x1j), 0.0)
      h = jnp.maximum(jnp.minimum(y2i, y2j) - jnp.maximum(y1i, y1j), 0.0)
      m = ((w * h) > 0.0) & (i_vec != j)
      return acc | m

    acc = lax.fori_loop(0, N_BOXES, jbody, jnp.zeros((L,), jnp.bool_))
    anyrow_v[...] = acc.astype(jnp.int32)
    pltpu.sync_copy(anyrow_v, anyov_sh.at[pl.ds(sid * L, L)])

  plsc.subcore_barrier()
  pltpu.sync_copy(anyov_sh, anyov_v)

  for g in range(PAIRS_PER_SUB // L):
    flat_base = (iota + g * L) * CPAD
    l0 = plsc.load_gather(lg_v, [flat_base])
    max1 = plsc.load_gather(lg_v, [flat_base + 1])

    def abody(c, carry):
      m1, am = carry
      lc = plsc.load_gather(lg_v, [flat_base + c])
      upd = lc > m1
      return jnp.where(upd, lc, m1), jnp.where(upd, c, am)

    max1, amax = lax.fori_loop(2, NUM_CLS, abody,
                               (max1, jnp.full((L,), 1, jnp.int32)))
    mrow = jnp.maximum(l0, max1)

    def sbody(c, s):
      lc = plsc.load_gather(lg_v, [flat_base + c])
      return s + jnp.exp(lc - mrow)

    ssum = lax.fori_loop(0, NUM_CLS, sbody, jnp.zeros((L,), jnp.float32))
    conf = jnp.exp(max1 - mrow) / ssum
    thr = plsc.load_gather(thr_v, [amax])

    av = a_v[pl.ds(g * L, L)]
    bv = b_v[pl.ds(g * L, L)]
    x1a = plsc.load_gather(x1_v, [av]); x1b = plsc.load_gather(x1_v, [bv])
    y1a = plsc.load_gather(y1_v, [av]); y1b = plsc.load_gather(y1_v, [bv])
    x2a = plsc.load_gather(x2_v, [av]); x2b = plsc.load_gather(x2_v, [bv])
    y2a = plsc.load_gather(y2_v, [av]); y2b = plsc.load_gather(y2_v, [bv])
    w = jnp.maximum(jnp.minimum(x2a, x2b) - jnp.maximum(x1a, x1b), 0.0)
    h = jnp.maximum(jnp.minimum(y2a, y2b) - jnp.maximum(y1a, y1b), 0.0)
    ovab = (w * h) > 0.0
    anyg = plsc.load_gather(anyov_v, [av])
    ovp = jnp.where(av == bv, anyg > 0, ovab)
    valid = (conf >= thr) & ovp

    conf_b[pl.ds(g * L, L)] = conf
    cls_b[pl.ds(g * L, L)] = amax
    valid_b[pl.ds(g * L, L)] = valid.astype(jnp.int32)

  pltpu.sync_copy(conf_b, conf_sh.at[pl.ds(sid * PAIRS_PER_SUB, PAIRS_PER_SUB)])
  pltpu.sync_copy(cls_b, cls_sh.at[pl.ds(sid * PAIRS_PER_SUB, PAIRS_PER_SUB)])
  pltpu.sync_copy(valid_b, valid_sh.at[pl.ds(sid * PAIRS_PER_SUB, PAIRS_PER_SUB)])
  plsc.subcore_barrier()

  pltpu.sync_copy(conf_sh, conf_all)
  pltpu.sync_copy(cls_sh, cls_all)
  pltpu.sync_copy(valid_sh, valid_all)

  t3c_acc = jnp.full((L,), -1.0, jnp.float32)
  t3i_acc = jnp.full((L,), BIG_I, jnp.int32)
  for cc in range(4):
    c = sid * 4 + cc
    excl1 = BIG_I
    excl2 = BIG_I
    for p in range(3):
      e1, e2 = excl1, excl2

      def kbody(k, carry):
        bc, bi = carry
        cf = conf_all[pl.ds(k * L, L)]
        cl = cls_all[pl.ds(k * L, L)]
        vd = valid_all[pl.ds(k * L, L)]
        idxv = iota + k * L
        m = (vd != 0) & (cl == c) & (idxv != e1) & (idxv != e2)
        upd = m & ((cf > bc) | ((cf == bc) & (idxv < bi)))
        return jnp.where(upd, cf, bc), jnp.where(upd, idxv, bi)

      bc, bi = lax.fori_loop(0, N_PAIRS // L, kbody,
                             (jnp.full((L,), -1.0, jnp.float32),
                              jnp.full((L,), BIG_I, jnp.int32)))
      mx = jnp.max(bc)
      i_star = jnp.min(jnp.where(bc == mx, bi, BIG_I))
      if p == 0:
        excl1 = i_star
      elif p == 1:
        excl2 = i_star
    t3c_acc = jnp.where(iota == cc, mx, t3c_acc)
    t3i_acc = jnp.where(iota == cc, i_star, t3i_acc)

  t3c_b[...] = t3c_acc
  t3i_b[...] = t3i_acc
  pltpu.sync_copy(t3c_b, t3c_sh.at[pl.ds(sid * L, L)])
  pltpu.sync_copy(t3i_b, t3i_sh.at[pl.ds(sid * L, L)])
  plsc.subcore_barrier()

  pltpu.sync_copy(t3c_sh, t3c_v)
  pltpu.sync_copy(t3i_sh, t3i_v)
  gbase = core * (N_PAIRS // NC) + sid * OUT_PER_SUB
  for g in range(OUT_PER_SUB // L):
    off = gbase + g * L
    cf = conf_all[pl.ds(off, L)]
    cl = cls_all[pl.ds(off, L)]
    vd = valid_all[pl.ds(off, L)]
    idxv = off + iota
    tidx = ((cl >> 2) << 4) | (cl & 3)
    c3 = plsc.load_gather(t3c_v, [tidx])
    i3 = plsc.load_gather(t3i_v, [tidx])
    keep = (vd != 0) & ((cf > c3) | ((cf == c3) & (idxv <= i3)))
    mc_b[pl.ds(g * L, L)] = cf * keep.astype(jnp.float32)
    fin_b[pl.ds(g * L, L)] = keep.astype(jnp.int32)
    cls_ob[pl.ds(g * L, L)] = cl
  pltpu.sync_copy(mc_b, mconf_hbm.at[pl.ds(gbase, OUT_PER_SUB)])
  pltpu.sync_copy(fin_b, final_hbm.at[pl.ds(gbase, OUT_PER_SUB)])
  pltpu.sync_copy(cls_ob, clso_hbm.at[pl.ds(gbase, OUT_PER_SUB)])


@functools.partial(
    pl.kernel,
    out_type=(jax.ShapeDtypeStruct((N_PAIRS,), jnp.float32),
              jax.ShapeDtypeStruct((N_PAIRS,), jnp.int32),
              jax.ShapeDtypeStruct((N_PAIRS,), jnp.int32)),
    mesh=plsc.VectorSubcoreMesh(core_axis_name="c", subcore_axis_name="s"),
    compiler_params=pltpu.CompilerParams(needs_layout_passes=False),
    scratch_types=(
        pltpu.VMEM((PAIRS_PER_SUB * CPAD,), jnp.float32),
        pltpu.VMEM((PAIRS_PER_SUB,), jnp.int32),
        pltpu.VMEM((PAIRS_PER_SUB,), jnp.int32),
        pltpu.VMEM((N_BOXES,), jnp.float32),
        pltpu.VMEM((N_BOXES,), jnp.float32),
        pltpu.VMEM((N_BOXES,), jnp.float32),
        pltpu.VMEM((N_BOXES,), jnp.float32),
        pltpu.VMEM((CPAD,), jnp.float32),
        pltpu.VMEM((L,), jnp.int32),
        pltpu.VMEM((N_BOXES,), jnp.int32),
        pltpu.VMEM((PAIRS_PER_SUB,), jnp.float32),
        pltpu.VMEM((PAIRS_PER_SUB,), jnp.int32),
        pltpu.VMEM((PAIRS_PER_SUB,), jnp.int32),
        pltpu.VMEM((N_PAIRS,), jnp.float32),
        pltpu.VMEM((N_PAIRS,), jnp.int32),
        pltpu.VMEM((N_PAIRS,), jnp.int32),
        pltpu.VMEM((L,), jnp.float32),
        pltpu.VMEM((L,), jnp.int32),
        pltpu.VMEM((NS * L,), jnp.float32),
        pltpu.VMEM((NS * L,), jnp.int32),
        pltpu.VMEM((OUT_PER_SUB,), jnp.float32),
        pltpu.VMEM((OUT_PER_SUB,), jnp.int32),
        pltpu.VMEM((OUT_PER_SUB,), jnp.int32),
        pltpu.VMEM_SHARED((N_BOXES,), jnp.int32),
        pltpu.VMEM_SHARED((N_PAIRS,), jnp.float32),
        pltpu.VMEM_SHARED((N_PAIRS,), jnp.int32),
        pltpu.VMEM_SHARED((N_PAIRS,), jnp.int32),
        pltpu.VMEM_SHARED((NS * L,), jnp.float32),
        pltpu.VMEM_SHARED((NS * L,), jnp.int32),
    ),
)
def _sgg_sc_kernel(*refs):
  _body(*refs)


def kernel(rel_pair_idxs, boxes, rel_labels, pred_rel_logits, pred_threshold):
  del rel_labels
  lg = jnp.pad(pred_rel_logits.astype(jnp.float32),
               ((0, 0), (0, CPAD - NUM_CLS))).reshape(-1)
  a = rel_pair_idxs[:, 0].astype(jnp.int32)
  b = rel_pair_idxs[:, 1].astype(jnp.int32)
  bx = boxes.astype(jnp.float32)
  thr = jnp.pad(pred_threshold.astype(jnp.float32), (0, CPAD - NUM_CLS))
  mconf, final, cls = _sgg_sc_kernel(
      lg, a, b, bx[:, 0], bx[:, 1], bx[:, 2], bx[:, 3], thr)
  return mconf, final, cls
